# baseline (speedup 1.0000x reference)
import functools

import jax
import jax.numpy as jnp
from jax import lax
from jax.experimental import pallas as pl
from jax.experimental.pallas import tpu as pltpu
from jax.experimental.pallas import tpu_sc as plsc

N = 10000
D = 128
E = 320000
NC = 2
NS = 16
NP = 10112
RPT = NP // NS
CH = 128
TCH = 2560
NC0 = TCH // NS
IBLK = 8

_BO = 128


def _mm_body(x_ref, w_ref, b_ref, o_ref):
    acc = jnp.dot(w_ref[...], x_ref[...], preferred_element_type=jnp.float32)
    o_ref[...] = acc + b_ref[...]


def _matmul(xT, W, b2):
    grid = (NP // _BO,)
    return pl.pallas_call(
        _mm_body,
        grid=grid,
        in_specs=[
            pl.BlockSpec((N, D), lambda i: (0, 0)),
            pl.BlockSpec((_BO, N), lambda i: (i, 0)),
            pl.BlockSpec((_BO, 1), lambda i: (i, 0)),
        ],
        out_specs=pl.BlockSpec((_BO, D), lambda i: (i, 0)),
        out_shape=jax.ShapeDtypeStruct((NP, D), jnp.float32),
    )(xT, W, b2)


def _sc_scatter(src_p, dst_p, nf_pad, zeros):
    mesh = plsc.VectorSubcoreMesh(core_axis_name="c", subcore_axis_name="s", num_cores=1)

    @functools.partial(
        pl.kernel,
        mesh=mesh,
        out_type=jax.ShapeDtypeStruct((NP, D), jnp.float32),
        scratch_types=[
            pltpu.VMEM((IBLK, CH), jnp.int32),
            pltpu.VMEM((IBLK, CH), jnp.int32),
            pltpu.VMEM((CH, D), jnp.float32),
            pltpu.VMEM((CH, D), jnp.float32),
            pltpu.VMEM_SHARED((NP, D), jnp.float32),
            pltpu.SemaphoreType.DMA,
            pltpu.SemaphoreType.DMA,
        ],
    )
    def body(src_hbm, dst_hbm, nf_hbm, z_hbm, out_hbm, sidx, didx, rows0,
             rows1, agg_sh, sem0, sem1):
        c = lax.axis_index("c")
        s = lax.axis_index("s")
        slab = pl.ds(s * RPT, RPT)

        @pl.when(c == 0)
        def _():
            pltpu.sync_copy(nf_hbm.at[slab], agg_sh.at[slab])

        plsc.subcore_barrier()

        cbase = s * NC0
        bufs = ((rows0, sem0), (rows1, sem1))

        def block(ib, carry):
            row = pl.multiple_of(cbase + ib * IBLK, IBLK)
            pltpu.sync_copy(src_hbm.at[pl.ds(row, IBLK)], sidx)
            pltpu.sync_copy(dst_hbm.at[pl.ds(row, IBLK)], didx)
            for b in range(2):
                pltpu.make_async_copy(nf_hbm.at[sidx.at[b]], bufs[b][0],
                                      bufs[b][1]).start()

            def step(j, c2):
                for bi in range(2):
                    i = j * 2 + bi
                    rows, sem = bufs[bi]
                    pltpu.make_async_copy(nf_hbm.at[sidx.at[i]], rows,
                                          sem).wait()
                    pltpu.sync_copy(rows, agg_sh.at[didx.at[i]], add=True)

                    @pl.when(i + 2 < IBLK)
                    def _():
                        pltpu.make_async_copy(nf_hbm.at[sidx.at[i + 2]], rows,
                                              sem).start()
                return c2

            lax.fori_loop(0, IBLK // 2, step, 0)
            return carry

        @pl.when(c == 0)
        def _():
            lax.fori_loop(0, NC0 // IBLK, block, 0)

        plsc.subcore_barrier()

        @pl.when(c == 0)
        def _():
            pltpu.sync_copy(agg_sh.at[slab], out_hbm.at[slab])

    return body(src_p, dst_p, nf_pad, zeros)


def _combine_body(p_ref, scale_ref, o_ref):
    o_ref[...] = (jnp.transpose(p_ref[...]) * scale_ref[0, 0])[None]


def _combine(p, scale):
    return pl.pallas_call(
        _combine_body,
        grid=(1,),
        in_specs=[
            pl.BlockSpec((N, D), lambda i: (0, 0)),
            pl.BlockSpec(memory_space=pltpu.SMEM),
        ],
        out_specs=pl.BlockSpec((1, D, N), lambda i: (0, 0, 0)),
        out_shape=jax.ShapeDtypeStruct((1, D, N), jnp.float32),
    )(p, scale)


def kernel(node_features, edge_mapping, batch_lens, W, b):
    x = node_features[0]
    xT = x.T
    b2 = b.reshape(N, 1)
    nf_pad = _matmul(xT, W, b2)

    src = edge_mapping[1]
    dst = edge_mapping[0]
    pad = TCH * CH - E
    src_p = jnp.concatenate([src, jnp.zeros((pad,), jnp.int32)])
    dst_p = jnp.concatenate(
        [dst, N + (jnp.arange(pad, dtype=jnp.int32) % (NP - N))])
    src_p = src_p.reshape(TCH, CH)
    dst_p = dst_p.reshape(TCH, CH)
    zeros = jnp.zeros((NP, D), jnp.float32)

    p = _sc_scatter(src_p, dst_p, nf_pad, zeros)

    scale = (1.0 / batch_lens).astype(jnp.float32).reshape(1, 1)
    return _combine(p, scale)


# speedup vs baseline: 1.1674x; 1.1674x over previous
import functools

import jax
import jax.numpy as jnp
from jax import lax
from jax.experimental import pallas as pl
from jax.experimental.pallas import tpu as pltpu
from jax.experimental.pallas import tpu_sc as plsc

N = 10000
D = 128
E = 320000
NC = 2
NS = 16
NP = 10112
RPT = NP // NS
CH = 128
RE = 125
TCH = 2560
NC0 = 80
NC1 = TCH // NS - NC0
IBLK = 8

_BO = 128


def _mm_body(x_ref, w_ref, b_ref, o_ref):
    acc = jnp.dot(w_ref[...], x_ref[...], preferred_element_type=jnp.float32)
    o_ref[...] = acc + b_ref[...]


def _matmul(xT, W, b2):
    grid = (NP // _BO,)
    return pl.pallas_call(
        _mm_body,
        grid=grid,
        in_specs=[
            pl.BlockSpec((N, D), lambda i: (0, 0)),
            pl.BlockSpec((_BO, N), lambda i: (i, 0)),
            pl.BlockSpec((_BO, 1), lambda i: (i, 0)),
        ],
        out_specs=pl.BlockSpec((_BO, D), lambda i: (i, 0)),
        out_shape=jax.ShapeDtypeStruct((NP, D), jnp.float32),
    )(xT, W, b2)


def _sc_scatter(src_p, dst_p, nf_pad, zeros):
    mesh = plsc.VectorSubcoreMesh(core_axis_name="c", subcore_axis_name="s")

    @functools.partial(
        pl.kernel,
        mesh=mesh,
        out_type=jax.ShapeDtypeStruct((NC, NP, D), jnp.float32),
        scratch_types=[
            pltpu.VMEM((IBLK, CH), jnp.int32),
            pltpu.VMEM((IBLK, CH), jnp.int32),
            pltpu.VMEM((CH, D), jnp.float32),
            pltpu.VMEM((CH, D), jnp.float32),
            pltpu.VMEM_SHARED((NP, D), jnp.float32),
            pltpu.SemaphoreType.DMA,
            pltpu.SemaphoreType.DMA,
        ],
    )
    def body(src_hbm, dst_hbm, nf_hbm, z_hbm, out_hbm, sidx, didx, rows0,
             rows1, agg_sh, sem0, sem1):
        c = lax.axis_index("c")
        s = lax.axis_index("s")
        slab = pl.ds(s * RPT, RPT)

        @pl.when(c == 0)
        def _():
            pltpu.sync_copy(nf_hbm.at[slab], agg_sh.at[slab])

        @pl.when(c == 1)
        def _():
            pltpu.sync_copy(z_hbm.at[slab], agg_sh.at[slab])

        plsc.subcore_barrier()

        nblk = jnp.where(c == 0, NC0 // IBLK, NC1 // IBLK)
        cbase = pl.multiple_of(
            jnp.where(c == 0, s * NC0, NS * NC0 + s * NC1), IBLK)

        bufs = ((rows0, sem0), (rows1, sem1))

        def block(ib, carry):
            row = pl.multiple_of(cbase + ib * IBLK, IBLK)
            pltpu.sync_copy(src_hbm.at[pl.ds(row, IBLK)], sidx)
            pltpu.sync_copy(dst_hbm.at[pl.ds(row, IBLK)], didx)
            for b in range(2):
                pltpu.make_async_copy(nf_hbm.at[sidx.at[b]], bufs[b][0],
                                      bufs[b][1]).start()

            def step(j, c2):
                for bi in range(2):
                    i = j * 2 + bi
                    rows, sem = bufs[bi]
                    pltpu.make_async_copy(nf_hbm.at[sidx.at[i]], rows,
                                          sem).wait()
                    pltpu.sync_copy(rows, agg_sh.at[didx.at[i]], add=True)

                    @pl.when(i + 2 < IBLK)
                    def _():
                        pltpu.make_async_copy(nf_hbm.at[sidx.at[i + 2]], rows,
                                              sem).start()
                return c2

            lax.fori_loop(0, IBLK // 2, step, 0)
            return carry

        lax.fori_loop(0, nblk, block, 0)
        plsc.subcore_barrier()
        pltpu.sync_copy(agg_sh.at[slab], out_hbm.at[c, slab])

    return body(src_p, dst_p, nf_pad, zeros)


def _combine_body(p0_ref, p1_ref, scale_ref, o_ref):
    a = p0_ref[0] + p1_ref[0]
    o_ref[...] = (jnp.transpose(a) * scale_ref[0, 0])[None]


def _combine(p, scale):
    return pl.pallas_call(
        _combine_body,
        grid=(1,),
        in_specs=[
            pl.BlockSpec((1, N, D), lambda i: (0, 0, 0)),
            pl.BlockSpec((1, N, D), lambda i: (1, 0, 0)),
            pl.BlockSpec(memory_space=pltpu.SMEM),
        ],
        out_specs=pl.BlockSpec((1, D, N), lambda i: (0, 0, 0)),
        out_shape=jax.ShapeDtypeStruct((1, D, N), jnp.float32),
    )(p, p, scale)


def _pad_edges(src, dst):
    src_r = src.reshape(TCH, RE)
    dst_r = dst.reshape(TCH, RE)
    ci = jnp.arange(TCH, dtype=jnp.int32)[:, None]
    k = jnp.arange(CH - RE, dtype=jnp.int32)[None, :]
    pad_src = jnp.zeros((TCH, CH - RE), jnp.int32)
    pad_dst = N + (ci * (CH - RE) + k) % (NP - N)
    return (jnp.concatenate([src_r, pad_src], axis=1),
            jnp.concatenate([dst_r, pad_dst.astype(jnp.int32)], axis=1))


def kernel(node_features, edge_mapping, batch_lens, W, b):
    x = node_features[0]
    xT = x.T
    b2 = b.reshape(N, 1)
    nf_pad = _matmul(xT, W, b2)

    src_p, dst_p = _pad_edges(edge_mapping[1], edge_mapping[0])
    zeros = jnp.zeros((NP, D), jnp.float32)

    p = _sc_scatter(src_p, dst_p, nf_pad, zeros)

    scale = (1.0 / batch_lens).astype(jnp.float32).reshape(1, 1)
    return _combine(p, scale)
